# Initial kernel scaffold; baseline (speedup 1.0000x reference)
import jax
import jax.numpy as jnp
from jax.experimental import pallas as pl


def kernel(input, weight):
    raise NotImplementedError("write your pallas kernel here")


# speedup vs baseline: 6.3694x; 6.3694x over previous
import functools

import jax
import jax.numpy as jnp
from jax import lax
from jax.experimental import pallas as pl
from jax.experimental.pallas import tpu as pltpu
from jax.experimental.pallas import tpu_sc as plsc

BATCH = 4096
HIST = 200
D_MODEL = 128

_NC = 2
_NS = 16
_NW = _NC * _NS
_B = BATCH * HIST
_BPW = _B // _NW
_C = 128
_NCH = _BPW // _C

_mesh = plsc.VectorSubcoreMesh(core_axis_name="c", subcore_axis_name="s")


@functools.partial(
    pl.kernel,
    mesh=_mesh,
    out_type=jax.ShapeDtypeStruct((_B, D_MODEL), jnp.float32),
    scratch_types=[
        pltpu.VMEM((_NCH, _C), jnp.int32),
        pltpu.VMEM((_C, D_MODEL), jnp.float32),
        pltpu.SemaphoreType.DMA,
    ],
)
def _emb_lookup(idx_hbm, table_hbm, out_hbm, idx_v, rows_v, sem):
    wid = lax.axis_index("s") * _NC + lax.axis_index("c")
    pltpu.sync_copy(idx_hbm.at[wid], idx_v)
    base = wid * _BPW

    def step(j, carry):
        pltpu.async_copy(table_hbm.at[idx_v.at[j]], rows_v, sem).wait()
        pltpu.sync_copy(rows_v, out_hbm.at[pl.ds(base + j * _C, _C)])
        return carry

    lax.fori_loop(0, _NCH, step, 0)


def kernel(input, weight):
    idx = input.reshape(_NW, _NCH, _C).astype(jnp.int32)
    out = _emb_lookup(idx, weight)
    return out.reshape(BATCH, HIST, D_MODEL)


# speedup vs baseline: 9.2051x; 1.4452x over previous
import functools

import jax
import jax.numpy as jnp
from jax import lax
from jax.experimental import pallas as pl
from jax.experimental.pallas import tpu as pltpu
from jax.experimental.pallas import tpu_sc as plsc

BATCH = 4096
HIST = 200
D_MODEL = 128

_NC = 2
_NS = 16
_NW = _NC * _NS
_B = BATCH * HIST
_BPW = _B // _NW
_C = 128
_NCH = _BPW // _C
_NBUF = 4
_NOUT = _NCH // _NBUF

_mesh = plsc.VectorSubcoreMesh(core_axis_name="c", subcore_axis_name="s")


@functools.partial(
    pl.kernel,
    mesh=_mesh,
    out_type=jax.ShapeDtypeStruct((_B, D_MODEL), jnp.float32),
    scratch_types=[
        pltpu.VMEM((_NCH, _C), jnp.int32),
        *[pltpu.VMEM((_C, D_MODEL), jnp.float32) for _ in range(_NBUF)],
        *[pltpu.SemaphoreType.DMA for _ in range(_NBUF)],
    ],
)
def _emb_lookup(idx_hbm, table_hbm, out_hbm, idx_v, *bufs_and_sems):
    rows = bufs_and_sems[:_NBUF]
    sems = bufs_and_sems[_NBUF:]
    wid = lax.axis_index("s") * _NC + lax.axis_index("c")
    pltpu.sync_copy(idx_hbm.at[wid], idx_v)
    base = wid * _BPW

    for b in range(_NBUF):
        pltpu.async_copy(table_hbm.at[idx_v.at[b]], rows[b], sems[b])

    def outer(i, carry):
        j0 = i * _NBUF
        for b in range(_NBUF):
            pltpu.make_async_copy(table_hbm.at[idx_v.at[b]], rows[b],
                                  sems[b]).wait()
            pltpu.sync_copy(rows[b],
                            out_hbm.at[pl.ds(base + (j0 + b) * _C, _C)])

            @pl.when(i < _NOUT - 1)
            def _():
                pltpu.async_copy(table_hbm.at[idx_v.at[j0 + b + _NBUF]],
                                 rows[b], sems[b])

        return carry

    lax.fori_loop(0, _NOUT, outer, 0)


def kernel(input, weight):
    idx = input.reshape(_NW, _NCH, _C).astype(jnp.int32)
    out = _emb_lookup(idx, weight)
    return out.reshape(BATCH, HIST, D_MODEL)


# speedup vs baseline: 9.2100x; 1.0005x over previous
import functools

import jax
import jax.numpy as jnp
from jax import lax
from jax.experimental import pallas as pl
from jax.experimental.pallas import tpu as pltpu
from jax.experimental.pallas import tpu_sc as plsc

BATCH = 4096
HIST = 200
D_MODEL = 128

_NC = 2
_NS = 16
_NW = _NC * _NS
_B = BATCH * HIST
_BPW = _B // _NW
_C = 64
_NCH = _BPW // _C
_NSLOT = 8
_PF = 4
_NOUT = _NCH // _NSLOT

_mesh = plsc.VectorSubcoreMesh(core_axis_name="c", subcore_axis_name="s")


@functools.partial(
    pl.kernel,
    mesh=_mesh,
    out_type=jax.ShapeDtypeStruct((_B, D_MODEL), jnp.float32),
    scratch_types=[
        pltpu.VMEM((_NCH, _C), jnp.int32),
        *[pltpu.VMEM((_C, D_MODEL), jnp.float32) for _ in range(_NSLOT)],
        *[pltpu.SemaphoreType.DMA for _ in range(2 * _NSLOT)],
    ],
)
def _emb_lookup(idx_hbm, table_hbm, out_hbm, idx_v, *rest):
    rows = rest[:_NSLOT]
    gsem = rest[_NSLOT:2 * _NSLOT]
    osem = rest[2 * _NSLOT:]
    wid = lax.axis_index("s") * _NC + lax.axis_index("c")
    pltpu.sync_copy(idx_hbm.at[wid], idx_v)
    base = wid * _BPW

    for j in range(_PF):
        pltpu.async_copy(table_hbm.at[idx_v.at[j]], rows[j], gsem[j])

    def outer(i, carry):
        j0 = i * _NSLOT
        for c in range(_NSLOT):
            j = j0 + c
            cn = (c + _PF) % _NSLOT
            pltpu.make_async_copy(table_hbm.at[idx_v.at[c]], rows[c],
                                  gsem[c]).wait()
            pltpu.async_copy(rows[c],
                             out_hbm.at[pl.ds(base + j * _C, _C)], osem[c])

            @pl.when(j >= _PF)
            def _():
                pltpu.make_async_copy(rows[cn],
                                      out_hbm.at[pl.ds(base, _C)],
                                      osem[cn]).wait()

            @pl.when(j + _PF < _NCH)
            def _():
                pltpu.async_copy(table_hbm.at[idx_v.at[j + _PF]],
                                 rows[cn], gsem[cn])

        return carry

    lax.fori_loop(0, _NOUT, outer, 0)

    for c in range(_PF):
        slot = (_NCH - _PF + c) % _NSLOT
        pltpu.make_async_copy(rows[slot], out_hbm.at[pl.ds(base, _C)],
                              osem[slot]).wait()


def kernel(input, weight):
    idx = input.reshape(_NW, _NCH, _C).astype(jnp.int32)
    out = _emb_lookup(idx, weight)
    return out.reshape(BATCH, HIST, D_MODEL)
